# Initial kernel scaffold; baseline (speedup 1.0000x reference)
import functools

import jax
import jax.numpy as jnp
from jax import lax
from jax.experimental import pallas as pl
from jax.experimental.pallas import tpu as pltpu
from jax.experimental.pallas import tpu_sc as plsc

N = 10000
F1 = 128
F2 = 64
NC = 2
NS = 16
NW = NC * NS
CHUNK = 128
ROW_BLK = 400
ROWS_PER_SUB = N // NS


def _make_spmm(e_pad, feat):
    per_w = e_pad // NW
    n_chunks = per_w // CHUNK
    nvec = feat // 16
    mesh = plsc.VectorSubcoreMesh(core_axis_name="c", subcore_axis_name="s")

    @functools.partial(
        pl.kernel,
        out_type=jax.ShapeDtypeStruct((NC, N, feat), jnp.float32),
        mesh=mesh,
        scratch_types=[
            pltpu.VMEM((CHUNK,), jnp.int32),
            pltpu.VMEM((CHUNK,), jnp.int32),
            pltpu.VMEM((CHUNK,), jnp.float32),
            pltpu.VMEM((CHUNK, feat), jnp.float32),
            pltpu.VMEM_SHARED((N, feat), jnp.float32),
            pltpu.SemaphoreType.DMA,
        ],
    )
    def spmm(table_hbm, src_hbm, dst_hbm, ew_hbm, zeros_hbm, out_hbm,
             src_v, dst_v, ew_v, rows_v, acc, sem):
        c = lax.axis_index("c")
        s = lax.axis_index("s")
        wid = c * NS + s
        rbase = s * ROWS_PER_SUB
        pltpu.sync_copy(zeros_hbm.at[pl.ds(rbase, ROWS_PER_SUB), :],
                        acc.at[pl.ds(rbase, ROWS_PER_SUB), :])
        plsc.subcore_barrier()

        ebase = wid * per_w

        def chunk_body(i, carry):
            off = ebase + i * CHUNK
            pltpu.sync_copy(src_hbm.at[pl.ds(off, CHUNK)], src_v)
            pltpu.sync_copy(dst_hbm.at[pl.ds(off, CHUNK)], dst_v)
            pltpu.sync_copy(ew_hbm.at[pl.ds(off, CHUNK)], ew_v)
            pltpu.async_copy(table_hbm.at[src_v], rows_v, sem).wait()

            def scale_body(e, carry2):
                w = plsc.load_gather(ew_v, [jnp.full((16,), e, jnp.int32)])
                for f in range(nvec):
                    rows_v[e, pl.ds(f * 16, 16)] = rows_v[e, pl.ds(f * 16, 16)] * w
                return carry2

            lax.fori_loop(0, CHUNK, scale_body, 0, unroll=2)
            pltpu.sync_copy(rows_v, acc.at[dst_v], add=True)
            return carry

        lax.fori_loop(0, n_chunks, chunk_body, 0)
        plsc.subcore_barrier()
        pltpu.sync_copy(acc.at[pl.ds(rbase, ROWS_PER_SUB), :],
                        out_hbm.at[c, pl.ds(rbase, ROWS_PER_SUB), :])

    return spmm


_spmm_f1 = None
_spmm_f2 = None


def _get_spmm(e_pad, feat):
    global _spmm_f1, _spmm_f2
    if feat == F1:
        if _spmm_f1 is None:
            _spmm_f1 = _make_spmm(e_pad, feat)
        return _spmm_f1
    if _spmm_f2 is None:
        _spmm_f2 = _make_spmm(e_pad, feat)
    return _spmm_f2


def _mm1(x, W1):
    def body(x_ref, w_ref, o_ref):
        o_ref[...] = jnp.dot(x_ref[...], w_ref[...],
                             preferred_element_type=jnp.float32)

    return pl.pallas_call(
        body,
        grid=(N // ROW_BLK,),
        in_specs=[
            pl.BlockSpec((ROW_BLK, F1), lambda i: (i, 0)),
            pl.BlockSpec((F1, F1), lambda i: (0, 0)),
        ],
        out_specs=pl.BlockSpec((ROW_BLK, F1), lambda i: (i, 0)),
        out_shape=jax.ShapeDtypeStruct((N, F1), jnp.float32),
    )(x, W1)


def _relu_mm2(p, b1, W2):
    def body(p_ref, b_ref, w_ref, o_ref):
        h = jax.nn.relu(p_ref[0] + p_ref[1] + b_ref[...])
        o_ref[...] = jnp.dot(h, w_ref[...], preferred_element_type=jnp.float32)

    return pl.pallas_call(
        body,
        grid=(N // ROW_BLK,),
        in_specs=[
            pl.BlockSpec((NC, ROW_BLK, F1), lambda i: (0, i, 0)),
            pl.BlockSpec((1, F1), lambda i: (0, 0)),
            pl.BlockSpec((F1, F2), lambda i: (0, 0)),
        ],
        out_specs=pl.BlockSpec((ROW_BLK, F2), lambda i: (i, 0)),
        out_shape=jax.ShapeDtypeStruct((N, F2), jnp.float32),
    )(p, b1.reshape(1, F1), W2)


def _logsoftmax_head(q, b2):
    def body(q_ref, b_ref, o_ref):
        z = q_ref[0] + q_ref[1] + b_ref[...]
        m = jnp.max(z, axis=1, keepdims=True)
        ez = jnp.exp(z - m)
        ssum = jnp.sum(ez, axis=1, keepdims=True)
        o_ref[...] = z - m - jnp.log(ssum)

    return pl.pallas_call(
        body,
        grid=(N // ROW_BLK,),
        in_specs=[
            pl.BlockSpec((NC, ROW_BLK, F2), lambda i: (0, i, 0)),
            pl.BlockSpec((1, F2), lambda i: (0, 0)),
        ],
        out_specs=pl.BlockSpec((ROW_BLK, F2), lambda i: (i, 0)),
        out_shape=jax.ShapeDtypeStruct((N, F2), jnp.float32),
    )(q, b2.reshape(1, F2))


@jax.jit
def kernel(x, edge_index, edge_weight, W1, b1, W2, b2):
    e = edge_weight.shape[0]
    e_pad = ((e + NW * CHUNK - 1) // (NW * CHUNK)) * (NW * CHUNK)
    pad = e_pad - e
    src = jnp.pad(edge_index[0].astype(jnp.int32), (0, pad))
    dst = jnp.pad(edge_index[1].astype(jnp.int32), (0, pad))
    ew = jnp.pad(edge_weight, (0, pad))

    zeros1 = jnp.zeros((N, F1), jnp.float32)
    zeros2 = jnp.zeros((N, F2), jnp.float32)

    support = _mm1(x, W1)
    p = _get_spmm(e_pad, F1)(support, src, dst, ew, zeros1)
    support2 = _relu_mm2(p, b1, W2)
    q = _get_spmm(e_pad, F2)(support2, src, dst, ew, zeros2)
    return _logsoftmax_head(q, b2)


# speedup vs baseline: 3.0915x; 3.0915x over previous
import functools

import jax
import jax.numpy as jnp
from jax import lax
from jax.experimental import pallas as pl
from jax.experimental.pallas import tpu as pltpu
from jax.experimental.pallas import tpu_sc as plsc

N = 10000
F1 = 128
F2 = 64
NC = 2
NS = 16
NW = NC * NS
CHUNK = 128
ROW_BLK = 400
SUB_ROWS = 624
TAIL_ROWS = N - NS * SUB_ROWS


def _bcast_lane(v16, e):
    return lax.gather(
        v16,
        jnp.full((16, 1), e, jnp.int32),
        lax.GatherDimensionNumbers(
            offset_dims=(), collapsed_slice_dims=(0,), start_index_map=(0,)),
        (1,),
        mode=lax.GatherScatterMode.PROMISE_IN_BOUNDS,
    )


def _make_spmm(e_pad, feat):
    per_w = e_pad // NW
    n_chunks = per_w // CHUNK
    nvec = feat // 16
    mesh = plsc.VectorSubcoreMesh(core_axis_name="c", subcore_axis_name="s")

    @functools.partial(
        pl.kernel,
        out_type=jax.ShapeDtypeStruct((NC, N, feat), jnp.float32),
        mesh=mesh,
        scratch_types=[
            pltpu.VMEM((CHUNK,), jnp.int32),
            pltpu.VMEM((CHUNK,), jnp.int32),
            pltpu.VMEM((CHUNK,), jnp.float32),
            pltpu.VMEM((CHUNK, feat), jnp.float32),
            pltpu.VMEM_SHARED((N, feat), jnp.float32),
            pltpu.SemaphoreType.DMA,
        ],
        compiler_params=pltpu.CompilerParams(use_tc_tiling_on_sc=False),
    )
    def spmm(table_hbm, src_hbm, dst_hbm, ew_hbm, zeros_hbm, out_hbm,
             src_v, dst_v, ew_v, rows_v, acc, sem):
        c = lax.axis_index("c")
        s = lax.axis_index("s")
        wid = c * NS + s
        rbase = s * SUB_ROWS
        pltpu.sync_copy(zeros_hbm.at[pl.ds(rbase, SUB_ROWS), :],
                        acc.at[pl.ds(rbase, SUB_ROWS), :])

        @pl.when(s == NS - 1)
        def _():
            pltpu.sync_copy(zeros_hbm.at[pl.ds(NS * SUB_ROWS, TAIL_ROWS), :],
                            acc.at[pl.ds(NS * SUB_ROWS, TAIL_ROWS), :])

        plsc.subcore_barrier()

        ebase = wid * per_w

        def chunk_body(i, carry):
            off = ebase + i * CHUNK
            pltpu.sync_copy(src_hbm.at[pl.ds(off, CHUNK)], src_v)
            pltpu.sync_copy(dst_hbm.at[pl.ds(off, CHUNK)], dst_v)
            pltpu.sync_copy(ew_hbm.at[pl.ds(off, CHUNK)], ew_v)
            pltpu.async_copy(table_hbm.at[src_v], rows_v, sem).wait()

            def group_body(g, carry2):
                w16 = ew_v[pl.ds(g * 16, 16)]
                for e in range(16):
                    w = _bcast_lane(w16, e)
                    row = g * 16 + e
                    for f in range(nvec):
                        rows_v[row, pl.ds(f * 16, 16)] = (
                            rows_v[row, pl.ds(f * 16, 16)] * w)
                return carry2

            lax.fori_loop(0, CHUNK // 16, group_body, 0)
            pltpu.sync_copy(rows_v, acc.at[dst_v], add=True)
            return carry

        lax.fori_loop(0, n_chunks, chunk_body, 0)
        plsc.subcore_barrier()
        pltpu.sync_copy(acc.at[pl.ds(rbase, SUB_ROWS), :],
                        out_hbm.at[c, pl.ds(rbase, SUB_ROWS), :])

        @pl.when(s == NS - 1)
        def _():
            pltpu.sync_copy(acc.at[pl.ds(NS * SUB_ROWS, TAIL_ROWS), :],
                            out_hbm.at[c, pl.ds(NS * SUB_ROWS, TAIL_ROWS), :])

    return spmm


_spmm_f1 = None
_spmm_f2 = None


def _get_spmm(e_pad, feat):
    global _spmm_f1, _spmm_f2
    if feat == F1:
        if _spmm_f1 is None:
            _spmm_f1 = _make_spmm(e_pad, feat)
        return _spmm_f1
    if _spmm_f2 is None:
        _spmm_f2 = _make_spmm(e_pad, feat)
    return _spmm_f2


def _mm1(x, W1):
    def body(x_ref, w_ref, o_ref):
        o_ref[...] = jnp.dot(x_ref[...], w_ref[...],
                             preferred_element_type=jnp.float32)

    return pl.pallas_call(
        body,
        grid=(N // ROW_BLK,),
        in_specs=[
            pl.BlockSpec((ROW_BLK, F1), lambda i: (i, 0)),
            pl.BlockSpec((F1, F1), lambda i: (0, 0)),
        ],
        out_specs=pl.BlockSpec((ROW_BLK, F1), lambda i: (i, 0)),
        out_shape=jax.ShapeDtypeStruct((N, F1), jnp.float32),
    )(x, W1)


def _relu_mm2(p, b1, W2):
    def body(p_ref, b_ref, w_ref, o_ref):
        h = jax.nn.relu(p_ref[0] + p_ref[1] + b_ref[...])
        o_ref[...] = jnp.dot(h, w_ref[...], preferred_element_type=jnp.float32)

    return pl.pallas_call(
        body,
        grid=(N // ROW_BLK,),
        in_specs=[
            pl.BlockSpec((NC, ROW_BLK, F1), lambda i: (0, i, 0)),
            pl.BlockSpec((1, F1), lambda i: (0, 0)),
            pl.BlockSpec((F1, F2), lambda i: (0, 0)),
        ],
        out_specs=pl.BlockSpec((ROW_BLK, F2), lambda i: (i, 0)),
        out_shape=jax.ShapeDtypeStruct((N, F2), jnp.float32),
    )(p, b1.reshape(1, F1), W2)


def _logsoftmax_head(q, b2):
    def body(q_ref, b_ref, o_ref):
        z = q_ref[0] + q_ref[1] + b_ref[...]
        m = jnp.max(z, axis=1, keepdims=True)
        ez = jnp.exp(z - m)
        ssum = jnp.sum(ez, axis=1, keepdims=True)
        o_ref[...] = z - m - jnp.log(ssum)

    return pl.pallas_call(
        body,
        grid=(N // ROW_BLK,),
        in_specs=[
            pl.BlockSpec((NC, ROW_BLK, F2), lambda i: (0, i, 0)),
            pl.BlockSpec((1, F2), lambda i: (0, 0)),
        ],
        out_specs=pl.BlockSpec((ROW_BLK, F2), lambda i: (i, 0)),
        out_shape=jax.ShapeDtypeStruct((N, F2), jnp.float32),
    )(q, b2.reshape(1, F2))


@jax.jit
def kernel(x, edge_index, edge_weight, W1, b1, W2, b2):
    e = edge_weight.shape[0]
    e_pad = ((e + NW * CHUNK - 1) // (NW * CHUNK)) * (NW * CHUNK)
    pad = e_pad - e
    src = jnp.pad(edge_index[0].astype(jnp.int32), (0, pad))
    dst = jnp.pad(edge_index[1].astype(jnp.int32), (0, pad))
    ew = jnp.pad(edge_weight, (0, pad))

    zeros1 = jnp.zeros((N, F1), jnp.float32)
    zeros2 = jnp.zeros((N, F2), jnp.float32)

    support = _mm1(x, W1)
    p = _get_spmm(e_pad, F1)(support, src, dst, ew, zeros1)
    support2 = _relu_mm2(p, b1, W2)
    q = _get_spmm(e_pad, F2)(support2, src, dst, ew, zeros2)
    return _logsoftmax_head(q, b2)
